# baseline (speedup 1.0000x reference)
import functools

import jax
import jax.numpy as jnp
from jax import lax
from jax.experimental import pallas as pl
from jax.experimental.pallas import tpu as pltpu
from jax.experimental.pallas import tpu_sc as plsc

N = 10000
E = 320000
D = 128
ED = 4
G = 64

C = 64
NW = 32
NSLOT = 6
KPW = 162
NQ = KPW // NSLOT
EPAD = NW * KPW * C
NSUB = 16
RPT = 624
RREM = N - NSUB * RPT


def _make_sc_layer():
    mesh = plsc.VectorSubcoreMesh(core_axis_name="c", subcore_axis_name="s")

    @functools.partial(
        pl.kernel,
        mesh=mesh,
        out_type=jax.ShapeDtypeStruct((2, N, D), jnp.float32),
        scratch_types=(
            [
                pltpu.VMEM((NSLOT, C, D), jnp.float32),
                pltpu.VMEM((NSLOT * C,), jnp.int32),
                pltpu.VMEM((NSLOT, C), jnp.int32),
                pltpu.VMEM_SHARED((N, D), jnp.float32),
            ]
            + [pltpu.SemaphoreType.DMA] * (3 * NSLOT)
        ),
    )
    def sc_layer(h_hbm, si_hbm, di_hbm, e_hbm, z_hbm, out_hbm,
                 rowsS, srcS, dstS, acc_sh, *sems):
        cid = lax.axis_index("c")
        sid = lax.axis_index("s")
        w = sid * 2 + cid
        rows = [rowsS.at[p] for p in range(NSLOT)]
        srcs = [srcS.at[pl.ds(p * C, C)] for p in range(NSLOT)]
        dsts = [dstS.at[p] for p in range(NSLOT)]
        sgs = sems[0:NSLOT]
        sss = sems[NSLOT:2 * NSLOT]
        sis = sems[2 * NSLOT:3 * NSLOT]

        r0 = sid * RPT
        pltpu.sync_copy(z_hbm.at[pl.ds(r0, RPT)], acc_sh.at[pl.ds(r0, RPT)])

        @pl.when(sid == 0)
        def _():
            pltpu.sync_copy(z_hbm.at[pl.ds(NSUB * RPT, RREM)],
                            acc_sh.at[pl.ds(NSUB * RPT, RREM)])

        plsc.subcore_barrier()

        ebase = w * KPW * C

        def is_real(k):
            return ebase + k * C < E

        def issue_idx(k, p):
            base = ebase + k * C
            pltpu.async_copy(si_hbm.at[pl.ds(base, C)], srcs[p], sis[p])
            pltpu.async_copy(di_hbm.at[pl.ds(base, C)], dsts[p], sis[p])
            pltpu.async_copy(e_hbm.at[pl.ds(base, C)], rows[p], sis[p])

        def wait_idx(p):
            pltpu.make_async_copy(si_hbm.at[pl.ds(0, C)], srcs[p], sis[p]).wait()
            pltpu.make_async_copy(di_hbm.at[pl.ds(0, C)], dsts[p], sis[p]).wait()
            pltpu.make_async_copy(e_hbm.at[pl.ds(0, C)], rows[p], sis[p]).wait()

        def issue_gather(p):
            pltpu.async_copy(h_hbm.at[srcs[p]], rows[p], sgs[p], add=True)

        def wait_gather(p):
            pltpu.make_async_copy(h_hbm.at[pl.ds(0, C)], rows[p], sgs[p]).wait()

        def issue_scatter(p):
            pltpu.async_copy(rows[p], acc_sh.at[dsts[p]], sss[p], add=True)

        def wait_scatter(p):
            pltpu.make_async_copy(h_hbm.at[pl.ds(0, C)], rows[p], sss[p]).wait()

        def compute(p):
            rp = rows[p]

            def edge_body(j, carry):
                for g in range(8):
                    hv = rp[j, pl.ds(g * 16, 16)]
                    rp[j, pl.ds(g * 16, 16)] = jnp.maximum(hv, 0.0)
                return carry

            lax.fori_loop(0, C, edge_body, 0, unroll=2)

        for k in (0, 1, 2):
            @pl.when(is_real(k))
            def _(k=k):
                issue_idx(k, k)

        for k in (0, 1):
            @pl.when(is_real(k))
            def _(k=k):
                wait_idx(k)
                issue_gather(k)

        def macro(kq, carry):
            j0 = NSLOT * kq
            for p in range(NSLOT):
                j = j0 + p

                @pl.when(is_real(j))
                def _(p=p):
                    wait_gather(p)
                    compute(p)
                    issue_scatter(p)

                @pl.when((j >= NSLOT - 3) & is_real(j - (NSLOT - 3)))
                def _(p=p):
                    wait_scatter((p + 3) % NSLOT)

                @pl.when(is_real(j + 3) & (j + 3 < KPW))
                def _(j=j, p=p):
                    issue_idx(j + 3, (p + 3) % NSLOT)

                @pl.when(is_real(j + 2) & (j + 2 < KPW))
                def _(p=p):
                    wait_idx((p + 2) % NSLOT)
                    issue_gather((p + 2) % NSLOT)

            return carry

        lax.fori_loop(0, NQ, macro, 0, unroll=False)

        for k in range(KPW - (NSLOT - 3), KPW):
            @pl.when(is_real(k))
            def _(k=k):
                wait_scatter(k % NSLOT)

        plsc.subcore_barrier()
        pltpu.sync_copy(acc_sh.at[pl.ds(r0, RPT)],
                        out_hbm.at[cid, pl.ds(r0, RPT)])

        @pl.when(sid == 0)
        def _():
            pltpu.sync_copy(acc_sh.at[pl.ds(NSUB * RPT, RREM)],
                            out_hbm.at[cid, pl.ds(NSUB * RPT, RREM)])

    return sc_layer


_sc_layer = _make_sc_layer()


def _proj_body(x_ref, w_ref, b_ref, o_ref):
    acc = lax.dot_general(x_ref[...], w_ref[...], (((1,), (0,)), ((), ())),
                          preferred_element_type=jnp.float32)
    o_ref[...] = jnp.maximum(acc + b_ref[...], 0.0)


def _proj(xp, wap, ba2):
    return pl.pallas_call(
        _proj_body,
        out_shape=jax.ShapeDtypeStruct((N, D), jnp.float32),
    )(xp, wap, ba2)


_EBLK = 7680


def _emb_body(a_ref, w_ref, b_ref, o_ref):
    o_ref[...] = lax.dot_general(a_ref[...], w_ref[...], (((1,), (0,)), ((), ())),
                                 preferred_element_type=jnp.float32) + b_ref[...]


def _emb(a, w, b2):
    return pl.pallas_call(
        _emb_body,
        grid=(EPAD // _EBLK,),
        in_specs=[
            pl.BlockSpec((_EBLK, ED), lambda i: (i, 0)),
            pl.BlockSpec((ED, D), lambda i: (0, 0)),
            pl.BlockSpec((1, D), lambda i: (0, 0)),
        ],
        out_specs=pl.BlockSpec((_EBLK, D), lambda i: (i, 0)),
        out_shape=jax.ShapeDtypeStruct((EPAD, D), jnp.float32),
    )(a, w, b2)


_BLK = 2000


def _mlp_core(h_ref, agg_ref, w1_ref, b1_ref, w2_ref, b2_ref):
    z = h_ref[...] + agg_ref[0] + agg_ref[1]
    z = jnp.maximum(
        lax.dot_general(z, w1_ref[...], (((1,), (0,)), ((), ())),
                        preferred_element_type=jnp.float32) + b1_ref[...], 0.0)
    z = lax.dot_general(z, w2_ref[...], (((1,), (0,)), ((), ())),
                        preferred_element_type=jnp.float32) + b2_ref[...]
    return jnp.maximum(z, 0.0)


def _mlp_body(h_ref, agg_ref, w1_ref, b1_ref, w2_ref, b2_ref, o_ref):
    o_ref[...] = _mlp_core(h_ref, agg_ref, w1_ref, b1_ref, w2_ref, b2_ref)


def _mlp(h, agg, w1, b12, w2, b22):
    return pl.pallas_call(
        _mlp_body,
        grid=(N // _BLK,),
        in_specs=[
            pl.BlockSpec((_BLK, D), lambda i: (i, 0)),
            pl.BlockSpec((2, _BLK, D), lambda i: (0, i, 0)),
            pl.BlockSpec((D, D), lambda i: (0, 0)),
            pl.BlockSpec((1, D), lambda i: (0, 0)),
            pl.BlockSpec((D, D), lambda i: (0, 0)),
            pl.BlockSpec((1, D), lambda i: (0, 0)),
        ],
        out_specs=pl.BlockSpec((_BLK, D), lambda i: (i, 0)),
        out_shape=jax.ShapeDtypeStruct((N, D), jnp.float32),
    )(h, agg, w1, b12, w2, b22)


def _pool_body(h_ref, b_ref, wh1_ref, bh1_ref, wh2_ref, bh2_ref, o_ref):
    bt = b_ref[...]
    io = lax.broadcasted_iota(jnp.int32, (G, N), 0)
    oht = (bt == io).astype(jnp.float32)
    g = lax.dot_general(oht, h_ref[...], (((1,), (0,)), ((), ())),
                        preferred_element_type=jnp.float32)
    q = jnp.maximum(
        lax.dot_general(g, wh1_ref[...], (((1,), (0,)), ((), ())),
                        preferred_element_type=jnp.float32) + bh1_ref[...], 0.0)
    o_ref[...] = lax.dot_general(q, wh2_ref[...], (((1,), (0,)), ((), ())),
                                 preferred_element_type=jnp.float32) + bh2_ref[...]


def _pool(h, batch2, wh1, bh12, wh2, bh22):
    return pl.pallas_call(
        _pool_body,
        out_shape=jax.ShapeDtypeStruct((G, 1), jnp.float32),
    )(h, batch2, wh1, bh12, wh2, bh22)


def kernel(x, edge_index, edge_attr, batch, Wa, ba, W1, b1, W2, b2,
           We1, be1, We2, be2, We3, be3, Wh1, bh1, Wh2, bh2):
    xp = jnp.pad(x, ((0, 0), (0, 16 - x.shape[1])))
    wap = jnp.pad(Wa, ((0, 16 - Wa.shape[0]), (0, 0)))
    h = _proj(xp, wap, ba.reshape(1, D))

    si = jnp.pad(edge_index[0], (0, EPAD - E))
    di = jnp.concatenate(
        [edge_index[1], N + (jnp.arange(EPAD - E, dtype=jnp.int32) % 8)])
    eap = jnp.pad(edge_attr, ((0, EPAD - E), (0, 0)))
    zeros = jnp.zeros((N, D), jnp.float32)
    b12 = b1.reshape(1, D)
    b22 = b2.reshape(1, D)

    es = [_emb(eap, We, be.reshape(1, D))
          for We, be in ((We1, be1), (We2, be2), (We3, be3))]

    agg = _sc_layer(h, si, di, es[0], zeros)
    h = _mlp(h, agg, W1, b12, W2, b22)
    agg = _sc_layer(h, si, di, es[1], zeros)
    h = _mlp(h, agg, W1, b12, W2, b22)
    agg = _sc_layer(h, si, di, es[2], zeros)
    h = _mlp(h, agg, W1, b12, W2, b22)
    out = _pool(h, batch.reshape(1, N), Wh1, bh1.reshape(1, G),
                Wh2, bh2.reshape(1, 1))
    return out.reshape(-1)


# speedup vs baseline: 1.0125x; 1.0125x over previous
import functools

import jax
import jax.numpy as jnp
from jax import lax
from jax.experimental import pallas as pl
from jax.experimental.pallas import tpu as pltpu
from jax.experimental.pallas import tpu_sc as plsc

N = 10000
E = 320000
D = 128
ED = 4
G = 64

C = 72
NW = 32
NSLOT = 5
KPW = 140
NQ = KPW // NSLOT
EPAD = NW * KPW * C
NP = N + 8
NSUB = 16
RPT = 624
RREM = N - NSUB * RPT
ZREM = NP - NSUB * RPT


def _make_sc_layer():
    mesh = plsc.VectorSubcoreMesh(core_axis_name="c", subcore_axis_name="s")

    @functools.partial(
        pl.kernel,
        mesh=mesh,
        out_type=jax.ShapeDtypeStruct((2, N, D), jnp.float32),
        scratch_types=(
            [
                pltpu.VMEM((NSLOT, C, D), jnp.float32),
                pltpu.VMEM((NSLOT, C), jnp.int32),
                pltpu.VMEM((NSLOT, C), jnp.int32),
                pltpu.VMEM_SHARED((NP, D), jnp.float32),
            ]
            + [pltpu.SemaphoreType.DMA] * (3 * NSLOT)
        ),
    )
    def sc_layer(h_hbm, si_hbm, di_hbm, e_hbm, z_hbm, out_hbm,
                 rowsS, srcS, dstS, acc_sh, *sems):
        cid = lax.axis_index("c")
        sid = lax.axis_index("s")
        w = sid * 2 + cid
        rows = [rowsS.at[p] for p in range(NSLOT)]
        srcs = [srcS.at[p] for p in range(NSLOT)]
        dsts = [dstS.at[p] for p in range(NSLOT)]
        sgs = sems[0:NSLOT]
        sss = sems[NSLOT:2 * NSLOT]
        sis = sems[2 * NSLOT:3 * NSLOT]

        r0 = sid * RPT
        pltpu.sync_copy(z_hbm.at[pl.ds(r0, RPT)], acc_sh.at[pl.ds(r0, RPT)])

        @pl.when(sid == 0)
        def _():
            pltpu.sync_copy(z_hbm.at[pl.ds(NSUB * RPT, ZREM)],
                            acc_sh.at[pl.ds(NSUB * RPT, ZREM)])

        plsc.subcore_barrier()

        ebase = w * KPW * C

        def is_real(k):
            return ebase + k * C < E

        def issue_idx(k, p):
            base = ebase + k * C
            pltpu.async_copy(si_hbm.at[pl.ds(base, C)], srcs[p], sis[p])
            pltpu.async_copy(di_hbm.at[pl.ds(base, C)], dsts[p], sis[p])
            pltpu.async_copy(e_hbm.at[pl.ds(base, C)], rows[p], sis[p])

        def wait_idx(p):
            pltpu.make_async_copy(si_hbm.at[pl.ds(0, C)], srcs[p], sis[p]).wait()
            pltpu.make_async_copy(di_hbm.at[pl.ds(0, C)], dsts[p], sis[p]).wait()
            pltpu.make_async_copy(e_hbm.at[pl.ds(0, C)], rows[p], sis[p]).wait()

        def issue_gather(p):
            pltpu.async_copy(h_hbm.at[srcs[p]], rows[p], sgs[p], add=True)

        def wait_gather(p):
            pltpu.make_async_copy(h_hbm.at[pl.ds(0, C)], rows[p], sgs[p]).wait()

        def issue_scatter(p):
            pltpu.async_copy(rows[p], acc_sh.at[dsts[p]], sss[p], add=True)

        def wait_scatter(p):
            pltpu.make_async_copy(h_hbm.at[pl.ds(0, C)], rows[p], sss[p]).wait()

        def compute(p):
            rp = rows[p]

            def edge_body(j, carry):
                for g in range(8):
                    hv = rp[j, pl.ds(g * 16, 16)]
                    rp[j, pl.ds(g * 16, 16)] = jnp.maximum(hv, 0.0)
                return carry

            lax.fori_loop(0, C, edge_body, 0, unroll=2)

        for k in (0, 1, 2):
            @pl.when(is_real(k))
            def _(k=k):
                issue_idx(k, k)

        for k in (0, 1):
            @pl.when(is_real(k))
            def _(k=k):
                wait_idx(k)
                issue_gather(k)

        def macro(kq, carry):
            j0 = NSLOT * kq
            for p in range(NSLOT):
                j = j0 + p

                @pl.when(is_real(j))
                def _(p=p):
                    wait_gather(p)
                    compute(p)
                    issue_scatter(p)

                @pl.when((j >= 2) & is_real(j - 2))
                def _(p=p):
                    wait_scatter((p + 3) % NSLOT)

                @pl.when(is_real(j + 3) & (j + 3 < KPW))
                def _(j=j, p=p):
                    issue_idx(j + 3, (p + 3) % NSLOT)

                @pl.when(is_real(j + 2) & (j + 2 < KPW))
                def _(p=p):
                    wait_idx((p + 2) % NSLOT)
                    issue_gather((p + 2) % NSLOT)

            return carry

        lax.fori_loop(0, NQ, macro, 0, unroll=False)

        for k in (KPW - 2, KPW - 1):
            @pl.when(is_real(k))
            def _(k=k):
                wait_scatter(k % NSLOT)

        plsc.subcore_barrier()
        pltpu.sync_copy(acc_sh.at[pl.ds(r0, RPT)],
                        out_hbm.at[cid, pl.ds(r0, RPT)])

        @pl.when(sid == 0)
        def _():
            pltpu.sync_copy(acc_sh.at[pl.ds(NSUB * RPT, RREM)],
                            out_hbm.at[cid, pl.ds(NSUB * RPT, RREM)])

    return sc_layer


_sc_layer = _make_sc_layer()


def _proj_body(x_ref, w_ref, b_ref, o_ref):
    acc = lax.dot_general(x_ref[...], w_ref[...], (((1,), (0,)), ((), ())),
                          preferred_element_type=jnp.float32)
    o_ref[...] = jnp.maximum(acc + b_ref[...], 0.0)


def _proj(xp, wap, ba2):
    return pl.pallas_call(
        _proj_body,
        out_shape=jax.ShapeDtypeStruct((N, D), jnp.float32),
    )(xp, wap, ba2)


_EBLK = 7680


def _emb_body(a_ref, w_ref, b_ref, o_ref):
    o_ref[...] = lax.dot_general(a_ref[...], w_ref[...], (((1,), (0,)), ((), ())),
                                 preferred_element_type=jnp.float32) + b_ref[...]


def _emb(a, w, b2):
    return pl.pallas_call(
        _emb_body,
        grid=(EPAD // _EBLK,),
        in_specs=[
            pl.BlockSpec((_EBLK, ED), lambda i: (i, 0)),
            pl.BlockSpec((ED, D), lambda i: (0, 0)),
            pl.BlockSpec((1, D), lambda i: (0, 0)),
        ],
        out_specs=pl.BlockSpec((_EBLK, D), lambda i: (i, 0)),
        out_shape=jax.ShapeDtypeStruct((EPAD, D), jnp.float32),
    )(a, w, b2)


_BLK = 2000


def _mlp_core(h_ref, agg_ref, w1_ref, b1_ref, w2_ref, b2_ref):
    z = h_ref[...] + agg_ref[0] + agg_ref[1]
    z = jnp.maximum(
        lax.dot_general(z, w1_ref[...], (((1,), (0,)), ((), ())),
                        preferred_element_type=jnp.float32) + b1_ref[...], 0.0)
    z = lax.dot_general(z, w2_ref[...], (((1,), (0,)), ((), ())),
                        preferred_element_type=jnp.float32) + b2_ref[...]
    return jnp.maximum(z, 0.0)


def _mlp_body(h_ref, agg_ref, w1_ref, b1_ref, w2_ref, b2_ref, o_ref):
    o_ref[...] = _mlp_core(h_ref, agg_ref, w1_ref, b1_ref, w2_ref, b2_ref)


def _mlp(h, agg, w1, b12, w2, b22):
    return pl.pallas_call(
        _mlp_body,
        grid=(N // _BLK,),
        in_specs=[
            pl.BlockSpec((_BLK, D), lambda i: (i, 0)),
            pl.BlockSpec((2, _BLK, D), lambda i: (0, i, 0)),
            pl.BlockSpec((D, D), lambda i: (0, 0)),
            pl.BlockSpec((1, D), lambda i: (0, 0)),
            pl.BlockSpec((D, D), lambda i: (0, 0)),
            pl.BlockSpec((1, D), lambda i: (0, 0)),
        ],
        out_specs=pl.BlockSpec((_BLK, D), lambda i: (i, 0)),
        out_shape=jax.ShapeDtypeStruct((N, D), jnp.float32),
    )(h, agg, w1, b12, w2, b22)


def _pool_body(h_ref, b_ref, wh1_ref, bh1_ref, wh2_ref, bh2_ref, o_ref):
    bt = b_ref[...]
    io = lax.broadcasted_iota(jnp.int32, (G, N), 0)
    oht = (bt == io).astype(jnp.float32)
    g = lax.dot_general(oht, h_ref[...], (((1,), (0,)), ((), ())),
                        preferred_element_type=jnp.float32)
    q = jnp.maximum(
        lax.dot_general(g, wh1_ref[...], (((1,), (0,)), ((), ())),
                        preferred_element_type=jnp.float32) + bh1_ref[...], 0.0)
    o_ref[...] = lax.dot_general(q, wh2_ref[...], (((1,), (0,)), ((), ())),
                                 preferred_element_type=jnp.float32) + bh2_ref[...]


def _pool(h, batch2, wh1, bh12, wh2, bh22):
    return pl.pallas_call(
        _pool_body,
        out_shape=jax.ShapeDtypeStruct((G, 1), jnp.float32),
    )(h, batch2, wh1, bh12, wh2, bh22)


def kernel(x, edge_index, edge_attr, batch, Wa, ba, W1, b1, W2, b2,
           We1, be1, We2, be2, We3, be3, Wh1, bh1, Wh2, bh2):
    xp = jnp.pad(x, ((0, 0), (0, 16 - x.shape[1])))
    wap = jnp.pad(Wa, ((0, 16 - Wa.shape[0]), (0, 0)))
    h = _proj(xp, wap, ba.reshape(1, D))

    si = jnp.pad(edge_index[0], (0, EPAD - E))
    di = jnp.concatenate(
        [edge_index[1], N + (jnp.arange(EPAD - E, dtype=jnp.int32) % 8)])
    eap = jnp.pad(edge_attr, ((0, EPAD - E), (0, 0)))
    zeros = jnp.zeros((NP, D), jnp.float32)
    b12 = b1.reshape(1, D)
    b22 = b2.reshape(1, D)

    es = [_emb(eap, We, be.reshape(1, D))
          for We, be in ((We1, be1), (We2, be2), (We3, be3))]

    agg = _sc_layer(h, si, di, es[0], zeros)
    h = _mlp(h, agg, W1, b12, W2, b22)
    agg = _sc_layer(h, si, di, es[1], zeros)
    h = _mlp(h, agg, W1, b12, W2, b22)
    agg = _sc_layer(h, si, di, es[2], zeros)
    h = _mlp(h, agg, W1, b12, W2, b22)
    out = _pool(h, batch.reshape(1, N), Wh1, bh1.reshape(1, G),
                Wh2, bh2.reshape(1, 1))
    return out.reshape(-1)
